# baseline (speedup 1.0000x reference)
import jax
import jax.numpy as jnp
from jax.experimental import pallas as pl
from jax.experimental.pallas import tpu as pltpu

RESV = 4096
NOUT = 128
BLK = 512
QR = NOUT // 4


def _body(x_ref, h_ref, wi_ref, wb_ref, wo0_ref, wo1_ref, wo2_ref, wo3_ref,
          w_hbm, o_ref, z_ref, wblk_ref, sem):
    x = x_ref[0]
    z_ref[...] = wi_ref[...] * x + wb_ref[...]
    nz = jnp.any(h_ref[...] != 0.0)

    @pl.when(nz)
    def _reservoir_matvec():
        def step(b, carry):
            cp = pltpu.make_async_copy(
                w_hbm.at[pl.ds(b * BLK, BLK), :], wblk_ref, sem)
            cp.start()
            cp.wait()
            mv = jax.lax.dot_general(
                h_ref[...], wblk_ref[...], (((0,), (1,)), ((), ())),
                preferred_element_type=jnp.float32)
            z_ref[pl.ds(b * BLK, BLK)] += mv
            return carry

        jax.lax.fori_loop(0, RESV // BLK, step, 0)

    t = jnp.tanh(z_ref[...])
    for q, wo_ref in enumerate((wo0_ref, wo1_ref, wo2_ref, wo3_ref)):
        o_ref[pl.ds(q * QR, QR)] = jax.lax.dot_general(
            wo_ref[...], t, (((1,), (0,)), ((), ())),
            preferred_element_type=jnp.float32)


def _quarter_spec(q):
    return pl.BlockSpec((QR, RESV), lambda i, q=q: (q, 0))


def kernel(x, W, W_input, W_bias, W_out, h):
    return pl.pallas_call(
        _body,
        grid=(1,),
        in_specs=[
            pl.BlockSpec((1,), lambda i: (0,)),
            pl.BlockSpec((RESV,), lambda i: (0,)),
            pl.BlockSpec((RESV,), lambda i: (0,)),
            pl.BlockSpec((RESV,), lambda i: (0,)),
            _quarter_spec(0),
            _quarter_spec(1),
            _quarter_spec(2),
            _quarter_spec(3),
            pl.BlockSpec(memory_space=pltpu.MemorySpace.HBM),
        ],
        out_specs=pl.BlockSpec((NOUT,), lambda i: (0,)),
        out_shape=jax.ShapeDtypeStruct((NOUT,), jnp.float32),
        scratch_shapes=[
            pltpu.VMEM((RESV,), jnp.float32),
            pltpu.VMEM((BLK, RESV), jnp.float32),
            pltpu.SemaphoreType.DMA,
        ],
    )(x, h, W_input, W_bias, W_out, W_out, W_out, W_out, W)
